# baseline (speedup 1.0000x reference)
import jax
import jax.numpy as jnp
from jax.experimental import pallas as pl
import jax.experimental.pallas.tpu as pltpu
from jax.experimental import layout as _layout

E = 64
TOP_K = 2
D_MODEL = 128
BATCH = 64
SEQ_LEN = 4096

CHUNK_B = 4
N_CH = BATCH // CHUNK_B
RING = 4


def _router_kernel(
    x_hbm, c_ref, scores_out_ref, idx_out_ref, acc_ref, sres_ref, ires_ref, out_sem, *rest
):
    bufs = rest[:RING]
    sems = rest[RING:]

    def copy(k):
        return pltpu.make_async_copy(
            x_hbm.at[pl.ds(k * CHUNK_B, CHUNK_B)], bufs[k % RING], sems[k % RING]
        )

    for k in range(RING):
        copy(k).start()
    for k in range(N_CH):
        copy(k).wait()
        acc_ref[pl.ds(k * CHUNK_B, CHUNK_B), :] = jnp.sum(bufs[k % RING][...], axis=1)
        if k + RING < N_CH:
            copy(k + RING).start()

    c = c_ref[...]
    cn = jnp.sqrt(jnp.sum(c * c, axis=1, keepdims=True))
    c = c / jnp.maximum(cn, 1e-12)

    q = acc_ref[...] * (1.0 / SEQ_LEN)
    qn = jnp.sqrt(jnp.sum(q * q, axis=1, keepdims=True))
    q = q / jnp.maximum(qn, 1e-12)

    scores = jax.lax.dot_general(
        q, c, (((1,), (1,)), ((), ())), preferred_element_type=jnp.float32
    )

    iota = jax.lax.broadcasted_iota(jnp.int32, (BATCH, E), 1)
    m1 = jnp.max(scores, axis=1, keepdims=True)
    i1 = jnp.min(
        jnp.where(scores == m1, iota, jnp.int32(2**30)), axis=1, keepdims=True
    )
    masked = jnp.where(iota == i1, -jnp.inf, scores)
    m2 = jnp.max(masked, axis=1, keepdims=True)
    i2 = jnp.min(
        jnp.where(masked == m2, iota, jnp.int32(2**30)), axis=1, keepdims=True
    )

    sres_ref[:, 0:1] = m1
    sres_ref[:, 1:2] = m2
    ires_ref[:, 0:1] = i1
    ires_ref[:, 1:2] = i2
    pltpu.make_async_copy(sres_ref, scores_out_ref, out_sem).start()
    pltpu.make_async_copy(ires_ref, idx_out_ref, out_sem).start()
    pltpu.make_async_copy(sres_ref, scores_out_ref, out_sem).wait()
    pltpu.make_async_copy(ires_ref, idx_out_ref, out_sem).wait()


def _kernel_impl(x, centroids):
    top_scores, top_idx = pl.pallas_call(
        _router_kernel,
        in_specs=[
            pl.BlockSpec(memory_space=pl.ANY),
            pl.BlockSpec(memory_space=pltpu.MemorySpace.VMEM),
        ],
        out_specs=[
            pl.BlockSpec(memory_space=pl.ANY),
            pl.BlockSpec(memory_space=pl.ANY),
        ],
        out_shape=[
            jax.ShapeDtypeStruct((BATCH, TOP_K), jnp.float32),
            jax.ShapeDtypeStruct((BATCH, TOP_K), jnp.int32),
        ],
        scratch_shapes=(
            [
                pltpu.VMEM((BATCH, D_MODEL), jnp.float32),
                pltpu.VMEM((BATCH, TOP_K), jnp.float32),
                pltpu.VMEM((BATCH, TOP_K), jnp.int32),
                pltpu.SemaphoreType.DMA,
            ]
            + [pltpu.VMEM((CHUNK_B, SEQ_LEN, D_MODEL), jnp.float32) for _ in range(RING)]
            + [pltpu.SemaphoreType.DMA for _ in range(RING)]
        ),
    )(x, centroids)
    return top_scores, top_idx


_kernel_impl.__name__ = "kernel"
_jitted = None


def kernel(x, centroids):
    global _jitted
    if _jitted is None:
        sharding = jax.sharding.SingleDeviceSharding(jax.devices()[0])
        linear = _layout.Format(_layout.Layout(major_to_minor=(1, 0)), sharding)
        _jitted = jax.jit(_kernel_impl, out_shardings=(linear, linear))
    return _jitted(x, centroids)


# speedup vs baseline: 1.0071x; 1.0071x over previous
import jax
import jax.numpy as jnp
from jax.experimental import pallas as pl
import jax.experimental.pallas.tpu as pltpu
from jax.experimental import layout as _layout

E = 64
TOP_K = 2
D_MODEL = 128
BATCH = 64
SEQ_LEN = 4096

CHUNK_B = 4
N_CH = BATCH // CHUNK_B
RING = 4


def _router_kernel(x_hbm, c_ref, out_ref, acc_ref, res_ref, out_sem, *rest):
    bufs = rest[:RING]
    sems = rest[RING:]

    def copy(k):
        return pltpu.make_async_copy(
            x_hbm.at[pl.ds(k * CHUNK_B, CHUNK_B)], bufs[k % RING], sems[k % RING]
        )

    for k in range(RING):
        copy(k).start()
    for k in range(N_CH):
        copy(k).wait()
        acc_ref[pl.ds(k * CHUNK_B, CHUNK_B), :] = jnp.sum(bufs[k % RING][...], axis=1)
        if k + RING < N_CH:
            copy(k + RING).start()

    c = c_ref[...]
    cn = jnp.sqrt(jnp.sum(c * c, axis=1, keepdims=True))
    c = c / jnp.maximum(cn, 1e-12)

    q = acc_ref[...] * (1.0 / SEQ_LEN)
    qn = jnp.sqrt(jnp.sum(q * q, axis=1, keepdims=True))
    q = q / jnp.maximum(qn, 1e-12)

    scores = jax.lax.dot_general(
        q, c, (((1,), (1,)), ((), ())), preferred_element_type=jnp.float32
    )

    iota = jax.lax.broadcasted_iota(jnp.int32, (BATCH, E), 1)
    m1 = jnp.max(scores, axis=1, keepdims=True)
    i1 = jnp.min(
        jnp.where(scores == m1, iota, jnp.int32(2**30)), axis=1, keepdims=True
    )
    masked = jnp.where(iota == i1, -jnp.inf, scores)
    m2 = jnp.max(masked, axis=1, keepdims=True)
    i2 = jnp.min(
        jnp.where(masked == m2, iota, jnp.int32(2**30)), axis=1, keepdims=True
    )

    res_ref[:, 0:1] = jax.lax.bitcast_convert_type(m1, jnp.int32)
    res_ref[:, 1:2] = jax.lax.bitcast_convert_type(m2, jnp.int32)
    res_ref[:, 2:3] = i1
    res_ref[:, 3:4] = i2
    pltpu.make_async_copy(res_ref, out_ref, out_sem).start()
    pltpu.make_async_copy(res_ref, out_ref, out_sem).wait()


def _kernel_impl(x, centroids):
    packed = pl.pallas_call(
        _router_kernel,
        in_specs=[
            pl.BlockSpec(memory_space=pl.ANY),
            pl.BlockSpec(memory_space=pltpu.MemorySpace.VMEM),
        ],
        out_specs=pl.BlockSpec(memory_space=pl.ANY),
        out_shape=jax.ShapeDtypeStruct((BATCH, 2 * TOP_K), jnp.int32),
        scratch_shapes=(
            [
                pltpu.VMEM((BATCH, D_MODEL), jnp.float32),
                pltpu.VMEM((BATCH, 2 * TOP_K), jnp.int32),
                pltpu.SemaphoreType.DMA,
            ]
            + [pltpu.VMEM((CHUNK_B, SEQ_LEN, D_MODEL), jnp.float32) for _ in range(RING)]
            + [pltpu.SemaphoreType.DMA for _ in range(RING)]
        ),
    )(x, centroids)
    top_scores = jax.lax.bitcast_convert_type(packed[:, 0:TOP_K], jnp.float32)
    top_idx = packed[:, TOP_K : 2 * TOP_K]
    return top_scores, top_idx


_kernel_impl.__name__ = "kernel"
_jitted = None


def kernel(x, centroids):
    global _jitted
    if _jitted is None:
        sharding = jax.sharding.SingleDeviceSharding(jax.devices()[0])
        linear = _layout.Format(_layout.Layout(major_to_minor=(1, 0)), sharding)
        _jitted = jax.jit(_kernel_impl, out_shardings=(linear, linear))
    return _jitted(x, centroids)
